# baseline (speedup 1.0000x reference)
import functools

import jax
import jax.numpy as jnp
from jax import lax
from jax.experimental import pallas as pl
from jax.experimental.pallas import tpu as pltpu
from jax.experimental.pallas import tpu_sc as plsc

N = 10000
H = 256
LANE_D, TYPE_D, LEN_D, NODE_D = 64, 32, 32, 128

_NC, _NS = 2, 16
_NW = _NC * _NS
_B_PAD = 10240
_BPW = _B_PAD // _NW
_CHUNK = 80

_BM = 1000
_BK = 1024
_NKB = (N + _BK - 1) // _BK


def _sc_gather_body(lane_i, type_i, len_i, node_i,
                    lane_t, type_t, len_t, node_t,
                    lane_o, type_o, len_o, node_o,
                    idx_v, rows_v, sem):
  wid = lax.axis_index("s") * _NC + lax.axis_index("c")
  base = wid * _BPW
  for i_hbm, t_hbm, o_hbm in (
      (lane_i, lane_t, lane_o),
      (type_i, type_t, type_o),
      (len_i, len_t, len_o),
      (node_i, node_t, node_o),
  ):
    pltpu.sync_copy(i_hbm.at[pl.ds(base, _BPW)], idx_v)
    for c in range(_BPW // _CHUNK):
      pltpu.async_copy(
          t_hbm.at[idx_v.at[pl.ds(c * _CHUNK, _CHUNK)]],
          rows_v.at[pl.ds(c * _CHUNK, _CHUNK)],
          sem,
      ).wait()
    pltpu.sync_copy(rows_v, o_hbm.at[pl.ds(base, _BPW)])


def _sc_gather(lane_i, type_i, len_i, node_i, lane_t, type_t, len_t, node_t):
  mesh = plsc.VectorSubcoreMesh(core_axis_name="c", subcore_axis_name="s")
  out = jax.ShapeDtypeStruct((_B_PAD, 128), jnp.float32)
  run = pl.kernel(
      _sc_gather_body,
      out_type=(out, out, out, out),
      mesh=mesh,
      scratch_types=[
          pltpu.VMEM((_BPW,), jnp.int32),
          pltpu.VMEM((_BPW, 128), jnp.float32),
          pltpu.SemaphoreType.DMA,
      ],
  )
  return run(lane_i, type_i, len_i, node_i, lane_t, type_t, len_t, node_t)


def _support1_body(lane_e, type_e, len_e, node_e, wl, wt, wn_, wd, out):
  acc = jnp.dot(lane_e[...].astype(jnp.bfloat16), wl[...],
                preferred_element_type=jnp.float32)
  acc += jnp.dot(type_e[...].astype(jnp.bfloat16), wt[...],
                 preferred_element_type=jnp.float32)
  acc += jnp.dot(len_e[...].astype(jnp.bfloat16), wn_[...],
                 preferred_element_type=jnp.float32)
  acc += jnp.dot(node_e[...].astype(jnp.bfloat16), wd[...],
                 preferred_element_type=jnp.float32)
  out[...] = acc


def _support1(lane_e, type_e, len_e, node_e, W):
  wb = W.astype(jnp.bfloat16)
  wl, wt, wn, wd = wb[:64], wb[64:96], wb[96:128], wb[128:]
  grid = (N // _BM,)
  return pl.pallas_call(
      _support1_body,
      grid=grid,
      in_specs=[
          pl.BlockSpec((_BM, LANE_D), lambda i: (i, 0)),
          pl.BlockSpec((_BM, TYPE_D), lambda i: (i, 0)),
          pl.BlockSpec((_BM, LEN_D), lambda i: (i, 0)),
          pl.BlockSpec((_BM, NODE_D), lambda i: (i, 0)),
          pl.BlockSpec((LANE_D, H), lambda i: (0, 0)),
          pl.BlockSpec((TYPE_D, H), lambda i: (0, 0)),
          pl.BlockSpec((LEN_D, H), lambda i: (0, 0)),
          pl.BlockSpec((NODE_D, H), lambda i: (0, 0)),
      ],
      out_specs=pl.BlockSpec((_BM, H), lambda i: (i, 0)),
      out_shape=jax.ShapeDtypeStruct((N, H), jnp.float32),
      compiler_params=pltpu.CompilerParams(
          dimension_semantics=("parallel",)),
  )(lane_e, type_e, len_e, node_e, wl, wt, wn, wd)


def _adj_layer_body(adj_ref, s_ref, b_ref, w_ref, out_ref, acc_ref, *,
                    fuse_w):
  k = pl.program_id(1)

  @pl.when(k == 0)
  def _():
    acc_ref[...] = jnp.zeros_like(acc_ref)

  adjb = adj_ref[...]
  sb = s_ref[...]
  @pl.when(k == _NKB - 1)
  def _():
    rem = N - (_NKB - 1) * _BK
    col = lax.broadcasted_iota(jnp.int32, (1, _BK), 1)
    row = lax.broadcasted_iota(jnp.int32, (_BK, 1), 0)
    acc_ref[...] += jnp.dot(
        jnp.where(col < rem, adj_ref[...], 0.0).astype(jnp.bfloat16),
        jnp.where(row < rem, s_ref[...], 0.0).astype(jnp.bfloat16),
        preferred_element_type=jnp.float32)

  @pl.when(k < _NKB - 1)
  def _():
    acc_ref[...] += jnp.dot(adj_ref[...].astype(jnp.bfloat16),
                            s_ref[...].astype(jnp.bfloat16),
                            preferred_element_type=jnp.float32)

  @pl.when(k == _NKB - 1)
  def _():
    y = jnp.maximum(acc_ref[...] + b_ref[...], 0.0)
    if fuse_w:
      out_ref[...] = jnp.dot(y.astype(jnp.bfloat16), w_ref[...],
                             preferred_element_type=jnp.float32)
    else:
      out_ref[...] = y


def _adj_layer(adj, s, b2d, W, fuse_w):
  grid = (N // _BM, _NKB)
  wb = (W if W is not None else jnp.zeros((H, H))).astype(jnp.bfloat16)
  return pl.pallas_call(
      functools.partial(_adj_layer_body, fuse_w=fuse_w),
      grid=grid,
      in_specs=[
          pl.BlockSpec((_BM, _BK), lambda i, k: (i, k)),
          pl.BlockSpec((_BK, H), lambda i, k: (k, 0)),
          pl.BlockSpec((1, H), lambda i, k: (0, 0)),
          pl.BlockSpec((H, H), lambda i, k: (0, 0)),
      ],
      out_specs=pl.BlockSpec((_BM, H), lambda i, k: (i, 0)),
      out_shape=jax.ShapeDtypeStruct((N, H), jnp.float32),
      scratch_shapes=[pltpu.VMEM((_BM, H), jnp.float32)],
      compiler_params=pltpu.CompilerParams(
          dimension_semantics=("parallel", "arbitrary")),
  )(adj, s, b2d, wb)


def kernel(node_feature, type_feature, length_feature, lane_feature, adj,
           node_table, type_table, length_table, lane_table, W, b):
  pad = _B_PAD - N

  def prep(idx):
    return jnp.pad(idx.astype(jnp.int32), (0, pad))

  def padt(t):
    return jnp.pad(t, ((0, 0), (0, 128 - t.shape[1])))

  lane_e, type_e, len_e, node_e = _sc_gather(
      prep(lane_feature), prep(type_feature), prep(length_feature),
      prep(node_feature),
      padt(lane_table), padt(type_table), padt(length_table), node_table)

  s1 = _support1(lane_e[:N, :LANE_D], type_e[:N, :TYPE_D],
                 len_e[:N, :LEN_D], node_e[:N], W)
  b2d = b.reshape(1, H)
  s2 = _adj_layer(adj, s1, b2d, W, fuse_w=True)
  x2 = _adj_layer(adj, s2, b2d, None, fuse_w=False)
  return x2


# speedup vs baseline: 1.0671x; 1.0671x over previous
import functools

import jax
import jax.numpy as jnp
from jax import lax
from jax.experimental import pallas as pl
from jax.experimental.pallas import tpu as pltpu
from jax.experimental.pallas import tpu_sc as plsc

N = 10000
H = 256
LANE_D, TYPE_D, LEN_D, NODE_D = 64, 32, 32, 128

_NC, _NS = 2, 16
_NW = _NC * _NS
_B_PAD = 10240
_BPW = _B_PAD // _NW
_CHUNK = 80

_BM = 1000
_BK = 1024
_NKB = (N + _BK - 1) // _BK


def _sc_gather_body(lane_i, type_i, len_i, node_i,
                    lane_t, type_t, len_t, node_t,
                    lane_o, type_o, len_o, node_o,
                    idx_v, rows_v, sem):
  wid = lax.axis_index("s") * _NC + lax.axis_index("c")
  base = wid * _BPW
  for i_hbm, t_hbm, o_hbm in (
      (lane_i, lane_t, lane_o),
      (type_i, type_t, type_o),
      (len_i, len_t, len_o),
      (node_i, node_t, node_o),
  ):
    pltpu.sync_copy(i_hbm.at[pl.ds(base, _BPW)], idx_v)
    for c in range(_BPW // _CHUNK):
      pltpu.async_copy(
          t_hbm.at[idx_v.at[pl.ds(c * _CHUNK, _CHUNK)]],
          rows_v.at[pl.ds(c * _CHUNK, _CHUNK)],
          sem,
      ).wait()
    pltpu.sync_copy(rows_v, o_hbm.at[pl.ds(base, _BPW)])


def _sc_gather(lane_i, type_i, len_i, node_i, lane_t, type_t, len_t, node_t):
  mesh = plsc.VectorSubcoreMesh(core_axis_name="c", subcore_axis_name="s")
  out = jax.ShapeDtypeStruct((_B_PAD, 128), jnp.float32)
  run = pl.kernel(
      _sc_gather_body,
      out_type=(out, out, out, out),
      mesh=mesh,
      scratch_types=[
          pltpu.VMEM((_BPW,), jnp.int32),
          pltpu.VMEM((_BPW, 128), jnp.float32),
          pltpu.SemaphoreType.DMA,
      ],
  )
  return run(lane_i, type_i, len_i, node_i, lane_t, type_t, len_t, node_t)


_DOT = functools.partial(jnp.dot, precision=lax.Precision.DEFAULT,
                         preferred_element_type=jnp.float32)


def _support1_body(lane_e, type_e, len_e, node_e, wl, wt, wn_, wd, out):
  acc = _DOT(lane_e[...], wl[...])
  acc += _DOT(type_e[...], wt[...])
  acc += _DOT(len_e[...], wn_[...])
  acc += _DOT(node_e[...], wd[...])
  out[...] = acc


def _support1(lane_e, type_e, len_e, node_e, W):
  wl, wt, wn, wd = W[:64], W[64:96], W[96:128], W[128:]
  grid = (N // _BM,)
  return pl.pallas_call(
      _support1_body,
      grid=grid,
      in_specs=[
          pl.BlockSpec((_BM, LANE_D), lambda i: (i, 0)),
          pl.BlockSpec((_BM, TYPE_D), lambda i: (i, 0)),
          pl.BlockSpec((_BM, LEN_D), lambda i: (i, 0)),
          pl.BlockSpec((_BM, NODE_D), lambda i: (i, 0)),
          pl.BlockSpec((LANE_D, H), lambda i: (0, 0)),
          pl.BlockSpec((TYPE_D, H), lambda i: (0, 0)),
          pl.BlockSpec((LEN_D, H), lambda i: (0, 0)),
          pl.BlockSpec((NODE_D, H), lambda i: (0, 0)),
      ],
      out_specs=pl.BlockSpec((_BM, H), lambda i: (i, 0)),
      out_shape=jax.ShapeDtypeStruct((N, H), jnp.float32),
      compiler_params=pltpu.CompilerParams(
          dimension_semantics=("parallel",)),
  )(lane_e, type_e, len_e, node_e, wl, wt, wn, wd)


_REM = N - (_NKB - 1) * _BK


def _adj_layer_body(adj_ref, s_ref, b_ref, w_ref, out_ref, acc_ref, *,
                    fuse_w):
  k = pl.program_id(1)

  @pl.when(k == 0)
  def _():
    acc_ref[...] = jnp.zeros_like(acc_ref)

  @pl.when(k < _NKB - 1)
  def _():
    sb = s_ref[pl.ds(k * _BK, _BK), :]
    acc_ref[...] += _DOT(adj_ref[...], sb)

  @pl.when(k == _NKB - 1)
  def _():
    acc_ref[...] += _DOT(adj_ref[:, :_REM],
                         s_ref[(_NKB - 1) * _BK:, :])
    y = jnp.maximum(acc_ref[...] + b_ref[...], 0.0)
    if fuse_w:
      out_ref[...] = _DOT(y, w_ref[...])
    else:
      out_ref[...] = y


def _adj_layer(adj, s, b2d, W, fuse_w):
  grid = (N // _BM, _NKB)
  w = W if W is not None else jnp.zeros((H, H), jnp.float32)
  return pl.pallas_call(
      functools.partial(_adj_layer_body, fuse_w=fuse_w),
      grid=grid,
      in_specs=[
          pl.BlockSpec((_BM, _BK), lambda i, k: (i, k)),
          pl.BlockSpec((N, H), lambda i, k: (0, 0)),
          pl.BlockSpec((1, H), lambda i, k: (0, 0)),
          pl.BlockSpec((H, H), lambda i, k: (0, 0)),
      ],
      out_specs=pl.BlockSpec((_BM, H), lambda i, k: (i, 0)),
      out_shape=jax.ShapeDtypeStruct((N, H), jnp.float32),
      scratch_shapes=[pltpu.VMEM((_BM, H), jnp.float32)],
      compiler_params=pltpu.CompilerParams(
          dimension_semantics=("parallel", "arbitrary")),
  )(adj, s, b2d, w)


def kernel(node_feature, type_feature, length_feature, lane_feature, adj,
           node_table, type_table, length_table, lane_table, W, b):
  pad = _B_PAD - N

  def prep(idx):
    return jnp.pad(idx.astype(jnp.int32), (0, pad))

  def padt(t):
    return jnp.pad(t, ((0, 0), (0, 128 - t.shape[1])))

  lane_e, type_e, len_e, node_e = _sc_gather(
      prep(lane_feature), prep(type_feature), prep(length_feature),
      prep(node_feature),
      padt(lane_table), padt(type_table), padt(length_table), node_table)

  s1 = _support1(lane_e[:N, :LANE_D], type_e[:N, :TYPE_D],
                 len_e[:N, :LEN_D], node_e[:N], W)
  b2d = b.reshape(1, H)
  s2 = _adj_layer(adj, s1, b2d, W, fuse_w=True)
  x2 = _adj_layer(adj, s2, b2d, None, fuse_w=False)
  return x2


# speedup vs baseline: 1.4091x; 1.3205x over previous
import functools

import jax
import jax.numpy as jnp
from jax import lax
from jax.experimental import pallas as pl
from jax.experimental.pallas import tpu as pltpu
from jax.experimental.pallas import tpu_sc as plsc

N = 10000
H = 256
LANE_K, TYPE_K, LEN_K = 10, 20, 100
NODE_D = 128

_NC, _NS = 2, 16
_NW = _NC * _NS
_B_PAD = 10240
_BPW = _B_PAD // _NW
_CHUNK = 80

_BM = 1000
_BK = 1024
_NKB = (N + _BK - 1) // _BK
_REM = N - (_NKB - 1) * _BK

_DOT = functools.partial(jnp.dot, precision=lax.Precision.DEFAULT,
                         preferred_element_type=jnp.float32)


def _sc_gather_body(idx_hbm, table_hbm, out_hbm, idx_v, rows_v, sem):
  wid = lax.axis_index("s") * _NC + lax.axis_index("c")
  base = wid * _BPW
  pltpu.sync_copy(idx_hbm.at[pl.ds(base, _BPW)], idx_v)
  handles = [
      pltpu.async_copy(
          table_hbm.at[idx_v.at[pl.ds(c * _CHUNK, _CHUNK)]],
          rows_v.at[pl.ds(c * _CHUNK, _CHUNK)],
          sem,
      )
      for c in range(_BPW // _CHUNK)
  ]
  for h in handles:
    h.wait()
  pltpu.sync_copy(rows_v, out_hbm.at[pl.ds(base, _BPW)])


def _sc_gather(node_idx, node_table):
  mesh = plsc.VectorSubcoreMesh(core_axis_name="c", subcore_axis_name="s")
  run = pl.kernel(
      _sc_gather_body,
      out_type=jax.ShapeDtypeStruct((_B_PAD, NODE_D), jnp.float32),
      mesh=mesh,
      scratch_types=[
          pltpu.VMEM((_BPW,), jnp.int32),
          pltpu.VMEM((_BPW, NODE_D), jnp.float32),
          pltpu.SemaphoreType.DMA,
      ],
  )
  return run(node_idx, node_table)


def _support1_body(node_e, lane_i, type_i, len_i,
                   lane_t, type_t, len_t, wl, wt, wn_, wd, out):
  def onehot(idx_col, k):
    cols = lax.broadcasted_iota(jnp.int32, (1, k), 1)
    return (idx_col == cols).astype(jnp.float32)

  acc = _DOT(node_e[...], wd[...])
  acc += _DOT(onehot(lane_i[...], 16), _DOT(lane_t[...], wl[...]))
  acc += _DOT(onehot(type_i[...], 24), _DOT(type_t[...], wt[...]))
  acc += _DOT(onehot(len_i[...], 104), _DOT(len_t[...], wn_[...]))
  out[...] = acc


def _support1(node_e, lane_i, type_i, len_i, lane_t, type_t, len_t, W):
  grid = (N // _BM,)
  return pl.pallas_call(
      _support1_body,
      grid=grid,
      in_specs=[
          pl.BlockSpec((_BM, NODE_D), lambda i: (i, 0)),
          pl.BlockSpec((_BM, 1), lambda i: (i, 0)),
          pl.BlockSpec((_BM, 1), lambda i: (i, 0)),
          pl.BlockSpec((_BM, 1), lambda i: (i, 0)),
          pl.BlockSpec((16, 64), lambda i: (0, 0)),
          pl.BlockSpec((24, 32), lambda i: (0, 0)),
          pl.BlockSpec((104, 32), lambda i: (0, 0)),
          pl.BlockSpec((64, H), lambda i: (0, 0)),
          pl.BlockSpec((32, H), lambda i: (0, 0)),
          pl.BlockSpec((32, H), lambda i: (0, 0)),
          pl.BlockSpec((NODE_D, H), lambda i: (0, 0)),
      ],
      out_specs=pl.BlockSpec((_BM, H), lambda i: (i, 0)),
      out_shape=jax.ShapeDtypeStruct((N, H), jnp.float32),
      compiler_params=pltpu.CompilerParams(
          dimension_semantics=("parallel",)),
  )(node_e, lane_i, type_i, len_i, lane_t, type_t, len_t,
    W[:64], W[64:96], W[96:128], W[128:])


def _adj_layer_body(adj_ref, s_ref, b_ref, w_ref, out_ref, acc_ref, *,
                    fuse_w):
  k = pl.program_id(1)

  @pl.when(k == 0)
  def _():
    acc_ref[...] = jnp.zeros_like(acc_ref)

  @pl.when(k < _NKB - 1)
  def _():
    sb = s_ref[pl.ds(k * _BK, _BK), :]
    acc_ref[...] += _DOT(adj_ref[...], sb)

  @pl.when(k == _NKB - 1)
  def _():
    acc_ref[...] += _DOT(adj_ref[:, :_REM],
                         s_ref[(_NKB - 1) * _BK:, :])
    y = jnp.maximum(acc_ref[...] + b_ref[...], 0.0)
    if fuse_w:
      out_ref[...] = _DOT(y, w_ref[...])
    else:
      out_ref[...] = y


def _adj_layer(adj, s, b2d, W, fuse_w):
  grid = (N // _BM, _NKB)
  w = W if W is not None else jnp.zeros((H, H), jnp.float32)
  return pl.pallas_call(
      functools.partial(_adj_layer_body, fuse_w=fuse_w),
      grid=grid,
      in_specs=[
          pl.BlockSpec((_BM, _BK), lambda i, k: (i, k)),
          pl.BlockSpec((N, H), lambda i, k: (0, 0)),
          pl.BlockSpec((1, H), lambda i, k: (0, 0)),
          pl.BlockSpec((H, H), lambda i, k: (0, 0)),
      ],
      out_specs=pl.BlockSpec((_BM, H), lambda i, k: (i, 0)),
      out_shape=jax.ShapeDtypeStruct((N, H), jnp.float32),
      scratch_shapes=[pltpu.VMEM((_BM, H), jnp.float32)],
      compiler_params=pltpu.CompilerParams(
          dimension_semantics=("parallel", "arbitrary")),
  )(adj, s, b2d, w)


def kernel(node_feature, type_feature, length_feature, lane_feature, adj,
           node_table, type_table, length_table, lane_table, W, b):
  node_idx = jnp.pad(node_feature.astype(jnp.int32), (0, _B_PAD - N))
  node_e = _sc_gather(node_idx, node_table)[:N]

  def padt(t, k_pad):
    return jnp.pad(t, ((0, k_pad - t.shape[0]), (0, 0)))

  col = lambda v: v.astype(jnp.int32).reshape(N, 1)
  s1 = _support1(node_e, col(lane_feature), col(type_feature),
                 col(length_feature), padt(lane_table, 16),
                 padt(type_table, 24), padt(length_table, 104), W)
  b2d = b.reshape(1, H)
  s2 = _adj_layer(adj, s1, b2d, W, fuse_w=True)
  x2 = _adj_layer(adj, s2, b2d, None, fuse_w=False)
  return x2


# speedup vs baseline: 1.4332x; 1.0171x over previous
import functools

import jax
import jax.numpy as jnp
from jax import lax
from jax.experimental import pallas as pl
from jax.experimental.pallas import tpu as pltpu
from jax.experimental.pallas import tpu_sc as plsc

N = 10000
H = 256
NODE_D = 128

_NC, _NS = 2, 16
_NW = _NC * _NS
_B_PAD = 10240
_BPW = _B_PAD // _NW
_CHUNK = 80

_BM = 1000
_BK = 1024
_NKB = (N + _BK - 1) // _BK
_REM = N - (_NKB - 1) * _BK

_DOT = functools.partial(jnp.dot, precision=lax.Precision.DEFAULT,
                         preferred_element_type=jnp.float32)


def _sc_gather_body(idx_hbm, table_hbm, out_hbm, idx_v, rows_v, sem):
  wid = lax.axis_index("s") * _NC + lax.axis_index("c")
  base = wid * _BPW
  pltpu.sync_copy(idx_hbm.at[pl.ds(base, _BPW)], idx_v)
  handles = [
      pltpu.async_copy(
          table_hbm.at[idx_v.at[pl.ds(c * _CHUNK, _CHUNK)]],
          rows_v.at[pl.ds(c * _CHUNK, _CHUNK)],
          sem,
      )
      for c in range(_BPW // _CHUNK)
  ]
  for h in handles:
    h.wait()
  pltpu.sync_copy(rows_v, out_hbm.at[pl.ds(base, _BPW)])


def _sc_gather(node_idx, node_table):
  mesh = plsc.VectorSubcoreMesh(core_axis_name="c", subcore_axis_name="s")
  run = pl.kernel(
      _sc_gather_body,
      out_type=jax.ShapeDtypeStruct((_B_PAD, NODE_D), jnp.float32),
      mesh=mesh,
      scratch_types=[
          pltpu.VMEM((_BPW,), jnp.int32),
          pltpu.VMEM((_BPW, NODE_D), jnp.float32),
          pltpu.SemaphoreType.DMA,
      ],
  )
  return run(node_idx, node_table)


def _gcn_body(adj_ref, node_e, lane_i, type_i, len_i,
              lane_t, type_t, len_t, wl, wt, wn_, wd, w_ref, b_ref,
              out_ref, acc_ref, s1_ref, s2_ref):
  l = pl.program_id(0)
  i = pl.program_id(1)
  k = pl.program_id(2)

  @pl.when(k == 0)
  def _():
    acc_ref[...] = jnp.zeros_like(acc_ref)

  @pl.when((l == 0) & (i == 0))
  def _():
    def onehot(idx_col, kk):
      cols = lax.broadcasted_iota(jnp.int32, (1, kk), 1)
      return (idx_col == cols).astype(jnp.float32)

    chunk = _DOT(node_e[...], wd[...])
    chunk += _DOT(onehot(lane_i[...], 16), _DOT(lane_t[...], wl[...]))
    chunk += _DOT(onehot(type_i[...], 24), _DOT(type_t[...], wt[...]))
    chunk += _DOT(onehot(len_i[...], 104), _DOT(len_t[...], wn_[...]))
    s1_ref[pl.ds(k * _BK, _BK), :] = chunk

  @pl.when((l == 0) & (k < _NKB - 1))
  def _():
    acc_ref[...] += _DOT(adj_ref[...], s1_ref[pl.ds(k * _BK, _BK), :])

  @pl.when((l == 1) & (k < _NKB - 1))
  def _():
    acc_ref[...] += _DOT(adj_ref[...], s2_ref[pl.ds(k * _BK, _BK), :])

  @pl.when(k == _NKB - 1)
  def _():
    @pl.when(l == 0)
    def _():
      acc_ref[...] += _DOT(adj_ref[:, :_REM], s1_ref[_TAIL:N, :])

    @pl.when(l == 1)
    def _():
      acc_ref[...] += _DOT(adj_ref[:, :_REM], s2_ref[_TAIL:, :])

    y = jnp.maximum(acc_ref[...] + b_ref[...], 0.0)

    @pl.when(l == 0)
    def _():
      s2_ref[pl.ds(i * _BM, _BM), :] = _DOT(y, w_ref[...])

    @pl.when(l == 1)
    def _():
      out_ref[...] = y


_TAIL = (_NKB - 1) * _BK


def _gcn(adj, node_e, lane_i, type_i, len_i, lane_t, type_t, len_t, W, b2d):
  grid = (2, N // _BM, _NKB)

  def pin_first(l, i, k):
    return (jnp.where((l == 0) & (i == 0), k, 0), 0)

  const = lambda l, i, k: (0, 0)
  return pl.pallas_call(
      _gcn_body,
      grid=grid,
      in_specs=[
          pl.BlockSpec((_BM, _BK), lambda l, i, k: (i, k)),
          pl.BlockSpec((_BK, NODE_D), pin_first),
          pl.BlockSpec((_BK, 1), pin_first),
          pl.BlockSpec((_BK, 1), pin_first),
          pl.BlockSpec((_BK, 1), pin_first),
          pl.BlockSpec((16, 64), const),
          pl.BlockSpec((24, 32), const),
          pl.BlockSpec((104, 32), const),
          pl.BlockSpec((64, H), const),
          pl.BlockSpec((32, H), const),
          pl.BlockSpec((32, H), const),
          pl.BlockSpec((NODE_D, H), const),
          pl.BlockSpec((H, H), const),
          pl.BlockSpec((1, H), const),
      ],
      out_specs=pl.BlockSpec((_BM, H),
                             lambda l, i, k: (jnp.where(l == 0, 0, i), 0)),
      out_shape=jax.ShapeDtypeStruct((N, H), jnp.float32),
      scratch_shapes=[
          pltpu.VMEM((_BM, H), jnp.float32),
          pltpu.VMEM((_NKB * _BK, H), jnp.float32),
          pltpu.VMEM((N, H), jnp.float32),
      ],
      compiler_params=pltpu.CompilerParams(
          dimension_semantics=("arbitrary", "arbitrary", "arbitrary")),
  )(adj, node_e, lane_i, type_i, len_i, lane_t, type_t, len_t,
    W[:64], W[64:96], W[96:128], W[128:], W, b2d)


def kernel(node_feature, type_feature, length_feature, lane_feature, adj,
           node_table, type_table, length_table, lane_table, W, b):
  node_idx = jnp.pad(node_feature.astype(jnp.int32), (0, _B_PAD - N))
  node_e = _sc_gather(node_idx, node_table)[:N]

  def padt(t, k_pad):
    return jnp.pad(t, ((0, k_pad - t.shape[0]), (0, 0)))

  col = lambda v: v.astype(jnp.int32).reshape(N, 1)
  return _gcn(adj, node_e, col(lane_feature), col(type_feature),
              col(length_feature), padt(lane_table, 16),
              padt(type_table, 24), padt(length_table, 104),
              W, b.reshape(1, H))


# speedup vs baseline: 1.7446x; 1.2172x over previous
import functools

import jax
import jax.numpy as jnp
from jax import lax
from jax.experimental import pallas as pl
from jax.experimental.pallas import tpu as pltpu
from jax.experimental.pallas import tpu_sc as plsc

N = 10000
H = 256
NODE_D = 128

_NC, _NS = 2, 16
_NW = _NC * _NS
_B_PAD = 10240
_BPW = _B_PAD // _NW
_CHUNK = 80

_BM = 2000
_BK = 1024
_NKB = (N + _BK - 1) // _BK
_REM = N - (_NKB - 1) * _BK

_DOT = functools.partial(jnp.dot, precision=lax.Precision.DEFAULT,
                         preferred_element_type=jnp.float32)


def _sc_gather_body(idx_hbm, table_hbm, out_hbm, idx_v, rows_v, sem):
  wid = lax.axis_index("s") * _NC + lax.axis_index("c")
  base = wid * _BPW
  pltpu.sync_copy(idx_hbm.at[pl.ds(base, _BPW)], idx_v)
  handles = [
      pltpu.async_copy(
          table_hbm.at[idx_v.at[pl.ds(c * _CHUNK, _CHUNK)]],
          rows_v.at[pl.ds(c * _CHUNK, _CHUNK)],
          sem,
      )
      for c in range(_BPW // _CHUNK)
  ]
  for h in handles:
    h.wait()
  pltpu.sync_copy(rows_v, out_hbm.at[pl.ds(base, _BPW)])


def _sc_gather(node_idx, node_table):
  mesh = plsc.VectorSubcoreMesh(core_axis_name="c", subcore_axis_name="s")
  run = pl.kernel(
      _sc_gather_body,
      out_type=jax.ShapeDtypeStruct((_B_PAD, NODE_D), jnp.float32),
      mesh=mesh,
      scratch_types=[
          pltpu.VMEM((_BPW,), jnp.int32),
          pltpu.VMEM((_BPW, NODE_D), jnp.float32),
          pltpu.SemaphoreType.DMA,
      ],
  )
  return run(node_idx, node_table)


def _gcn_body(adj_ref, node_e, lane_i, type_i, len_i,
              lane_t, type_t, len_t, wl, wt, wn_, wd, w_ref, b_ref,
              out_ref, acc_ref, s1_ref, s2_ref):
  l = pl.program_id(0)
  i = pl.program_id(1)
  k = pl.program_id(2)

  @pl.when(k == 0)
  def _():
    acc_ref[...] = jnp.zeros_like(acc_ref)

  @pl.when((l == 0) & (i == 0))
  def _():
    def onehot(idx_col, kk):
      cols = lax.broadcasted_iota(jnp.int32, (1, kk), 1)
      return (idx_col == cols).astype(jnp.float32)

    chunk = _DOT(node_e[...], wd[...])
    chunk += _DOT(onehot(lane_i[...], 16), _DOT(lane_t[...], wl[...]))
    chunk += _DOT(onehot(type_i[...], 24), _DOT(type_t[...], wt[...]))
    chunk += _DOT(onehot(len_i[...], 104), _DOT(len_t[...], wn_[...]))
    s1_ref[pl.ds(k * _BK, _BK), :] = chunk

  @pl.when((l == 0) & (k < _NKB - 1))
  def _():
    acc_ref[...] += _DOT(adj_ref[...], s1_ref[pl.ds(k * _BK, _BK), :])

  @pl.when((l == 1) & (k < _NKB - 1))
  def _():
    acc_ref[...] += _DOT(adj_ref[...], s2_ref[pl.ds(k * _BK, _BK), :])

  @pl.when(k == _NKB - 1)
  def _():
    @pl.when(l == 0)
    def _():
      acc_ref[...] += _DOT(adj_ref[:, :_REM], s1_ref[_TAIL:N, :])

    @pl.when(l == 1)
    def _():
      acc_ref[...] += _DOT(adj_ref[:, :_REM], s2_ref[_TAIL:, :])

    y = jnp.maximum(acc_ref[...] + b_ref[...], 0.0)

    @pl.when(l == 0)
    def _():
      s2_ref[pl.ds(i * _BM, _BM), :] = _DOT(y, w_ref[...])

    @pl.when(l == 1)
    def _():
      out_ref[...] = y


_TAIL = (_NKB - 1) * _BK


def _gcn(adj, node_e, lane_i, type_i, len_i, lane_t, type_t, len_t, W, b2d):
  grid = (2, N // _BM, _NKB)

  def pin_first(l, i, k):
    return (jnp.where((l == 0) & (i == 0), k, 0), 0)

  const = lambda l, i, k: (0, 0)
  return pl.pallas_call(
      _gcn_body,
      grid=grid,
      in_specs=[
          pl.BlockSpec((_BM, _BK), lambda l, i, k: (i, k)),
          pl.BlockSpec((_BK, NODE_D), pin_first),
          pl.BlockSpec((_BK, 1), pin_first),
          pl.BlockSpec((_BK, 1), pin_first),
          pl.BlockSpec((_BK, 1), pin_first),
          pl.BlockSpec((16, 64), const),
          pl.BlockSpec((24, 32), const),
          pl.BlockSpec((104, 32), const),
          pl.BlockSpec((64, H), const),
          pl.BlockSpec((32, H), const),
          pl.BlockSpec((32, H), const),
          pl.BlockSpec((NODE_D, H), const),
          pl.BlockSpec((H, H), const),
          pl.BlockSpec((1, H), const),
      ],
      out_specs=pl.BlockSpec((_BM, H),
                             lambda l, i, k: (jnp.where(l == 0, 0, i), 0)),
      out_shape=jax.ShapeDtypeStruct((N, H), jnp.float32),
      scratch_shapes=[
          pltpu.VMEM((_BM, H), jnp.float32),
          pltpu.VMEM((_NKB * _BK, H), jnp.float32),
          pltpu.VMEM((N, H), jnp.float32),
      ],
      compiler_params=pltpu.CompilerParams(
          dimension_semantics=("arbitrary", "arbitrary", "arbitrary")),
  )(adj, node_e, lane_i, type_i, len_i, lane_t, type_t, len_t,
    W[:64], W[64:96], W[96:128], W[128:], W, b2d)


def kernel(node_feature, type_feature, length_feature, lane_feature, adj,
           node_table, type_table, length_table, lane_table, W, b):
  node_idx = jnp.pad(node_feature.astype(jnp.int32), (0, _B_PAD - N))
  node_e = _sc_gather(node_idx, node_table)[:N]

  def padt(t, k_pad):
    return jnp.pad(t, ((0, k_pad - t.shape[0]), (0, 0)))

  col = lambda v: v.astype(jnp.int32).reshape(N, 1)
  return _gcn(adj, node_e, col(lane_feature), col(type_feature),
              col(length_feature), padt(lane_table, 16),
              padt(type_table, 24), padt(length_table, 104),
              W, b.reshape(1, H))


# speedup vs baseline: 1.7989x; 1.0311x over previous
import functools

import jax
import jax.numpy as jnp
from jax import lax
from jax.experimental import pallas as pl
from jax.experimental.pallas import tpu as pltpu
from jax.experimental.pallas import tpu_sc as plsc

N = 10000
H = 256
NODE_D = 128

_NC, _NS = 2, 16
_NW = _NC * _NS
_B_PAD = 10240
_BPW = _B_PAD // _NW
_CHUNK = 80

_BM = 2000
_BK = 1280
_NKB = (N + _BK - 1) // _BK
_REM = N - (_NKB - 1) * _BK

_DOT = functools.partial(jnp.dot, precision=lax.Precision.DEFAULT,
                         preferred_element_type=jnp.float32)


def _sc_gather_body(idx_hbm, table_hbm, out_hbm, idx_v, rows_v, sem):
  wid = lax.axis_index("s") * _NC + lax.axis_index("c")
  base = wid * _BPW
  pltpu.sync_copy(idx_hbm.at[pl.ds(base, _BPW)], idx_v)
  handles = [
      pltpu.async_copy(
          table_hbm.at[idx_v.at[pl.ds(c * _CHUNK, _CHUNK)]],
          rows_v.at[pl.ds(c * _CHUNK, _CHUNK)],
          sem,
      )
      for c in range(_BPW // _CHUNK)
  ]
  for h in handles:
    h.wait()
  pltpu.sync_copy(rows_v, out_hbm.at[pl.ds(base, _BPW)])


def _sc_gather(node_idx, node_table):
  mesh = plsc.VectorSubcoreMesh(core_axis_name="c", subcore_axis_name="s")
  run = pl.kernel(
      _sc_gather_body,
      out_type=jax.ShapeDtypeStruct((_B_PAD, NODE_D), jnp.float32),
      mesh=mesh,
      scratch_types=[
          pltpu.VMEM((_BPW,), jnp.int32),
          pltpu.VMEM((_BPW, NODE_D), jnp.float32),
          pltpu.SemaphoreType.DMA,
      ],
  )
  return run(node_idx, node_table)


def _gcn_body(adj_ref, node_e, lane_i, type_i, len_i,
              lane_t, type_t, len_t, wl, wt, wn_, wd, w_ref, b_ref,
              out_ref, acc_ref, s1_ref, s2_ref):
  l = pl.program_id(0)
  i = pl.program_id(1)
  k = pl.program_id(2)

  @pl.when(k == 0)
  def _():
    acc_ref[...] = jnp.zeros_like(acc_ref)

  @pl.when((l == 0) & (i == 0))
  def _():
    def onehot(idx_col, kk):
      cols = lax.broadcasted_iota(jnp.int32, (1, kk), 1)
      return (idx_col == cols).astype(jnp.float32)

    chunk = _DOT(node_e[...], wd[...])
    chunk += _DOT(onehot(lane_i[...], 16), _DOT(lane_t[...], wl[...]))
    chunk += _DOT(onehot(type_i[...], 24), _DOT(type_t[...], wt[...]))
    chunk += _DOT(onehot(len_i[...], 104), _DOT(len_t[...], wn_[...]))

    @pl.when(k < _NKB - 1)
    def _():
      s1_ref[pl.ds(k * _BK, _BK), :] = chunk

    @pl.when(k == _NKB - 1)
    def _():
      s1_ref[_TAIL:, :] = chunk[:_REM, :]

  @pl.when((l == 0) & (k < _NKB - 1))
  def _():
    acc_ref[...] += _DOT(adj_ref[...], s1_ref[pl.ds(k * _BK, _BK), :])

  @pl.when((l == 1) & (k < _NKB - 1))
  def _():
    acc_ref[...] += _DOT(adj_ref[...], s2_ref[pl.ds(k * _BK, _BK), :])

  @pl.when(k == _NKB - 1)
  def _():
    @pl.when(l == 0)
    def _():
      acc_ref[...] += _DOT(adj_ref[:, :_REM], s1_ref[_TAIL:, :])

    @pl.when(l == 1)
    def _():
      acc_ref[...] += _DOT(adj_ref[:, :_REM], s2_ref[_TAIL:, :])

    y = jnp.maximum(acc_ref[...] + b_ref[...], 0.0)

    @pl.when(l == 0)
    def _():
      s2_ref[pl.ds(i * _BM, _BM), :] = _DOT(y, w_ref[...])

    @pl.when(l == 1)
    def _():
      out_ref[...] = y


_TAIL = (_NKB - 1) * _BK


def _gcn(adj, node_e, lane_i, type_i, len_i, lane_t, type_t, len_t, W, b2d):
  grid = (2, N // _BM, _NKB)

  def pin_first(l, i, k):
    return (jnp.where((l == 0) & (i == 0), k, 0), 0)

  const = lambda l, i, k: (0, 0)
  return pl.pallas_call(
      _gcn_body,
      grid=grid,
      in_specs=[
          pl.BlockSpec((_BM, _BK), lambda l, i, k: (i, k)),
          pl.BlockSpec((_BK, NODE_D), pin_first),
          pl.BlockSpec((_BK, 1), pin_first),
          pl.BlockSpec((_BK, 1), pin_first),
          pl.BlockSpec((_BK, 1), pin_first),
          pl.BlockSpec((16, 64), const),
          pl.BlockSpec((24, 32), const),
          pl.BlockSpec((104, 32), const),
          pl.BlockSpec((64, H), const),
          pl.BlockSpec((32, H), const),
          pl.BlockSpec((32, H), const),
          pl.BlockSpec((NODE_D, H), const),
          pl.BlockSpec((H, H), const),
          pl.BlockSpec((1, H), const),
      ],
      out_specs=pl.BlockSpec((_BM, H),
                             lambda l, i, k: (jnp.where(l == 0, 0, i), 0)),
      out_shape=jax.ShapeDtypeStruct((N, H), jnp.float32),
      scratch_shapes=[
          pltpu.VMEM((_BM, H), jnp.float32),
          pltpu.VMEM((N, H), jnp.float32),
          pltpu.VMEM((N, H), jnp.float32),
      ],
      compiler_params=pltpu.CompilerParams(
          dimension_semantics=("arbitrary", "arbitrary", "arbitrary")),
  )(adj, node_e, lane_i, type_i, len_i, lane_t, type_t, len_t,
    W[:64], W[64:96], W[96:128], W[128:], W, b2d)


def kernel(node_feature, type_feature, length_feature, lane_feature, adj,
           node_table, type_table, length_table, lane_table, W, b):
  node_idx = jnp.pad(node_feature.astype(jnp.int32), (0, _B_PAD - N))
  node_e = _sc_gather(node_idx, node_table)[:N]

  def padt(t, k_pad):
    return jnp.pad(t, ((0, k_pad - t.shape[0]), (0, 0)))

  col = lambda v: v.astype(jnp.int32).reshape(N, 1)
  return _gcn(adj, node_e, col(lane_feature), col(type_feature),
              col(length_feature), padt(lane_table, 16),
              padt(type_table, 24), padt(length_table, 104),
              W, b.reshape(1, H))
